# baseline (speedup 1.0000x reference)
import jax
import jax.numpy as jnp
from jax import lax
from jax.experimental import pallas as pl
from jax.experimental.pallas import tpu as pltpu

C = 256
G = 32
CPG = 8
H = 384
W = 384
D = 2
EPS = 1e-5
HSTEP = 8
N_GROUP = float(CPG * H * W)
N_SPATIAL = float(H * W)


def _fused_kernel(x_ref, sp_ref, bp_ref, o_ref, sa_ref, sb_ref, sems):
    def _cp(ch, d, dst):
        return pltpu.make_async_copy(
            x_ref.at[0, ch, :, d, :], dst.at[ch], sems.at[ch, d])

    for ch in range(CPG):
        _cp(ch, 0, sa_ref).start()
        _cp(ch, 1, sb_ref).start()

    tot_q0 = jnp.zeros((HSTEP, W), jnp.float32)
    tot_q1 = jnp.zeros((HSTEP, W), jnp.float32)
    tot_p = jnp.zeros((HSTEP, W), jnp.float32)
    c0_rows = []
    c1_rows = []
    for ch in range(CPG):
        a_s = jnp.zeros((HSTEP, W), jnp.float32)
        b_s = jnp.zeros((HSTEP, W), jnp.float32)
        a_q = jnp.zeros((HSTEP, W), jnp.float32)
        b_q = jnp.zeros((HSTEP, W), jnp.float32)
        p_q = jnp.zeros((HSTEP, W), jnp.float32)
        _cp(ch, 0, sa_ref).wait()
        _cp(ch, 1, sb_ref).wait()
        for j in range(0, H, HSTEP):
            av = sa_ref[ch, j:j + HSTEP]
            bv = sb_ref[ch, j:j + HSTEP]
            a_s = a_s + av
            b_s = b_s + bv
            a_q = a_q + av * av
            b_q = b_q + bv * bv
            p_q = p_q + av * bv
        tot_q0 = tot_q0 + a_q
        tot_q1 = tot_q1 + b_q
        tot_p = tot_p + p_q
        c0_rows.append(jnp.sum(a_s, axis=0, keepdims=True))
        c1_rows.append(jnp.sum(b_s, axis=0, keepdims=True))

    C0 = jnp.concatenate(c0_rows, axis=0)
    C1 = jnp.concatenate(c1_rows, axis=0)
    cs0 = jnp.sum(C0, axis=1, keepdims=True)
    cs1 = jnp.sum(C1, axis=1, keepdims=True)
    s0 = jnp.sum(cs0)
    s1 = jnp.sum(cs1)
    q00 = jnp.sum(tot_q0)
    q11 = jnp.sum(tot_q1)
    q01 = jnp.sum(tot_p)

    inv_n = 1.0 / N_GROUP
    m0 = s0 * inv_n
    m1 = s1 * inv_n
    a = q00 * inv_n - m0 * m0 + EPS
    c = q11 * inv_n - m1 * m1 + EPS
    b = q01 * inv_n - m0 * m1
    det = jnp.maximum(a * c - b * b, 1e-30)
    s = jnp.sqrt(det)
    t = a + c + 2.0 * s
    inv = lax.rsqrt(t) / s
    w00 = (c + s) * inv
    w11 = (a + s) * inv
    w01 = -b * inv

    sp = sp_ref[0]
    bp = bp_ref[0]
    inv_sp = 1.0 / N_SPATIAL
    for ch in range(CPG):
        sc = sp[ch, 0]
        bi = bp[ch, 0]
        p00 = sc * w00
        p01 = sc * w01
        p11 = sc * w11
        mv0 = cs0[ch, 0] * inv_sp
        mv1 = cs1[ch, 0] * inv_sp
        pq0 = bi * mv0 - p00 * m0 - p01 * m1
        pq1 = bi * mv1 - p01 * m0 - p11 * m1
        for j in range(0, H, HSTEP):
            av = sa_ref[ch, j:j + HSTEP]
            bv = sb_ref[ch, j:j + HSTEP]
            o_ref[0, ch, j:j + HSTEP, 0, :] = p00 * av + p01 * bv + pq0
            o_ref[0, ch, j:j + HSTEP, 1, :] = p01 * av + p11 * bv + pq1


def _compiler_params(**kw):
    cp = getattr(pltpu, "CompilerParams", None) or pltpu.TPUCompilerParams
    return cp(**kw)


def kernel(x, scale, bias):
    xt = jnp.transpose(x, (0, 1, 3, 2)).reshape(G, CPG, H, D, W)
    sp = jnp.broadcast_to(scale.reshape(G, CPG, 1), (G, CPG, W))
    bp = jnp.broadcast_to(bias.reshape(G, CPG, 1), (G, CPG, W))

    big_spec = pl.BlockSpec((1, CPG, H, D, W), lambda g: (g, 0, 0, 0, 0))
    chan_spec = pl.BlockSpec((1, CPG, W), lambda g: (g, 0, 0))

    out = pl.pallas_call(
        _fused_kernel,
        grid=(G,),
        in_specs=[big_spec] + [chan_spec] * 2,
        out_specs=big_spec,
        out_shape=jax.ShapeDtypeStruct((G, CPG, H, D, W), jnp.float32),
        scratch_shapes=[pltpu.VMEM((CPG, H, W), jnp.float32),
                        pltpu.VMEM((CPG, H, W), jnp.float32),
                        pltpu.SemaphoreType.DMA((CPG, D))],
        compiler_params=_compiler_params(
            dimension_semantics=("parallel",),
            vmem_limit_bytes=58_000_000),
    )(xt, sp, bp)
    return jnp.transpose(out.reshape(C, H, D, W), (0, 1, 3, 2))


# speedup vs baseline: 1.1721x; 1.1721x over previous
import jax
import jax.numpy as jnp
from jax import lax
from jax.experimental import pallas as pl
from jax.experimental.pallas import tpu as pltpu

C = 256
G = 32
CPG = 8
H = 384
W = 384
D = 2
EPS = 1e-5
HSTEP = 8
N_GROUP = float(CPG * H * W)
N_SPATIAL = float(H * W)


def _fused_kernel(x_ref, sp_ref, bp_ref, o_ref, sa_ref, sb_ref):
    tot_q0 = jnp.zeros((HSTEP, W), jnp.float32)
    tot_q1 = jnp.zeros((HSTEP, W), jnp.float32)
    tot_p = jnp.zeros((HSTEP, W), jnp.float32)
    c0_rows = []
    c1_rows = []
    for ch in range(CPG):
        a_s = jnp.zeros((HSTEP, W), jnp.float32)
        b_s = jnp.zeros((HSTEP, W), jnp.float32)
        a_q = jnp.zeros((HSTEP, W), jnp.float32)
        b_q = jnp.zeros((HSTEP, W), jnp.float32)
        p_q = jnp.zeros((HSTEP, W), jnp.float32)
        for j in range(0, H, HSTEP):
            xs = x_ref[0, ch, j:j + HSTEP]
            av = xs[:, 0, :]
            bv = xs[:, 1, :]
            sa_ref[ch, j:j + HSTEP] = av
            sb_ref[ch, j:j + HSTEP] = bv
            a_s = a_s + av
            b_s = b_s + bv
            a_q = a_q + av * av
            b_q = b_q + bv * bv
            p_q = p_q + av * bv
        tot_q0 = tot_q0 + a_q
        tot_q1 = tot_q1 + b_q
        tot_p = tot_p + p_q
        c0_rows.append(jnp.sum(a_s, axis=0, keepdims=True))
        c1_rows.append(jnp.sum(b_s, axis=0, keepdims=True))

    C0 = jnp.concatenate(c0_rows, axis=0)
    C1 = jnp.concatenate(c1_rows, axis=0)
    cs0 = jnp.sum(C0, axis=1, keepdims=True)
    cs1 = jnp.sum(C1, axis=1, keepdims=True)
    s0 = jnp.sum(cs0)
    s1 = jnp.sum(cs1)
    q00 = jnp.sum(tot_q0)
    q11 = jnp.sum(tot_q1)
    q01 = jnp.sum(tot_p)

    inv_n = 1.0 / N_GROUP
    m0 = s0 * inv_n
    m1 = s1 * inv_n
    a = q00 * inv_n - m0 * m0 + EPS
    c = q11 * inv_n - m1 * m1 + EPS
    b = q01 * inv_n - m0 * m1
    det = jnp.maximum(a * c - b * b, 1e-30)
    s = jnp.sqrt(det)
    t = a + c + 2.0 * s
    inv = lax.rsqrt(t) / s
    w00 = (c + s) * inv
    w11 = (a + s) * inv
    w01 = -b * inv

    sp = sp_ref[0]
    bp = bp_ref[0]
    inv_sp = 1.0 / N_SPATIAL
    for ch in range(CPG):
        sc = sp[ch, 0]
        bi = bp[ch, 0]
        p00 = sc * w00
        p01 = sc * w01
        p11 = sc * w11
        mv0 = cs0[ch, 0] * inv_sp
        mv1 = cs1[ch, 0] * inv_sp
        pq0 = bi * mv0 - p00 * m0 - p01 * m1
        pq1 = bi * mv1 - p01 * m0 - p11 * m1
        for j in range(0, H, HSTEP):
            av = sa_ref[ch, j:j + HSTEP]
            bv = sb_ref[ch, j:j + HSTEP]
            o_ref[0, ch, j:j + HSTEP, 0, :] = p00 * av + p01 * bv + pq0
            o_ref[0, ch, j:j + HSTEP, 1, :] = p01 * av + p11 * bv + pq1


def _compiler_params(**kw):
    cp = getattr(pltpu, "CompilerParams", None) or pltpu.TPUCompilerParams
    return cp(**kw)


def kernel(x, scale, bias):
    xt = jnp.transpose(x, (0, 1, 3, 2)).reshape(G, CPG, H, D, W)
    sp = jnp.broadcast_to(scale.reshape(G, CPG, 1), (G, CPG, W))
    bp = jnp.broadcast_to(bias.reshape(G, CPG, 1), (G, CPG, W))

    big_spec = pl.BlockSpec((1, CPG, H, D, W), lambda g: (g, 0, 0, 0, 0))
    chan_spec = pl.BlockSpec((1, CPG, W), lambda g: (g, 0, 0))

    out = pl.pallas_call(
        _fused_kernel,
        grid=(G,),
        in_specs=[big_spec] + [chan_spec] * 2,
        out_specs=big_spec,
        out_shape=jax.ShapeDtypeStruct((G, CPG, H, D, W), jnp.float32),
        scratch_shapes=[pltpu.VMEM((CPG, H, W), jnp.float32),
                        pltpu.VMEM((CPG, H, W), jnp.float32)],
        compiler_params=_compiler_params(
            dimension_semantics=("parallel",),
            vmem_limit_bytes=58_000_000),
    )(xt, sp, bp)
    return jnp.transpose(out.reshape(C, H, D, W), (0, 1, 3, 2))
